# baseline (speedup 1.0000x reference)
import jax
import jax.numpy as jnp
from jax.experimental import pallas as pl
from jax.experimental.pallas import tpu as pltpu

_B, _S = 8, 4
_T = _B * _S
_HIDDEN = 512
_INTER = 1024
_E = 8
_K = 2


def _moe_kernel(x_ref, rw_ref, w13_ref, w2_ref, out_ref, dw_ref):
    e = pl.program_id(0)
    x = x_ref[...]
    cols = jax.lax.broadcasted_iota(jnp.int32, (_T, _E), 1)

    @pl.when(e == 0)
    def _():
        logits = jnp.dot(x, rw_ref[...].T,
                         preferred_element_type=jnp.float32)
        m = jnp.max(logits, axis=-1, keepdims=True)
        ex = jnp.exp(logits - m)
        probs = ex / jnp.sum(ex, axis=-1, keepdims=True)

        i1 = jnp.argmax(probs, axis=-1, keepdims=True)
        v1 = jnp.max(probs, axis=-1)
        masked = jnp.where(cols == i1, -1.0, probs)
        i2 = jnp.argmax(masked, axis=-1, keepdims=True)
        v2 = jnp.max(masked, axis=-1)
        denom = v1 + v2 + 1e-6
        sel = (cols == i1) | (cols == i2)
        dw_ref[...] = jnp.where(sel, probs, 0.0) / denom[:, None]

    w_e = jnp.sum(jnp.where(cols == e, dw_ref[...], 0.0), axis=-1)

    h13 = jnp.dot(x, w13_ref[0], preferred_element_type=jnp.float32)
    gate = h13[:, :_INTER]
    up = h13[:, _INTER:]
    h = (gate * jax.nn.sigmoid(gate)) * up
    out_e = jnp.dot(h, w2_ref[0], preferred_element_type=jnp.float32)

    contrib = out_e * w_e[:, None]

    @pl.when(e == 0)
    def _():
        out_ref[...] = contrib

    @pl.when(e != 0)
    def _():
        out_ref[...] = out_ref[...] + contrib


@jax.jit
def kernel(x, router_w, w13, w2):
    xt = x.reshape(_T, _HIDDEN)
    out = pl.pallas_call(
        _moe_kernel,
        grid=(_E,),
        in_specs=[
            pl.BlockSpec((_T, _HIDDEN), lambda e: (0, 0)),
            pl.BlockSpec((_E, _HIDDEN), lambda e: (0, 0)),
            pl.BlockSpec((1, _HIDDEN, 2 * _INTER), lambda e: (e, 0, 0)),
            pl.BlockSpec((1, _INTER, _HIDDEN), lambda e: (e, 0, 0)),
        ],
        out_specs=pl.BlockSpec((_T, _HIDDEN), lambda e: (0, 0)),
        out_shape=jax.ShapeDtypeStruct((_T, _HIDDEN), jnp.float32),
        scratch_shapes=[pltpu.VMEM((_T, _E), jnp.float32)],
    )(xt, router_w, w13, w2)
    return out.reshape(_B, _S, _HIDDEN)


# speedup vs baseline: 1.0189x; 1.0189x over previous
import jax
import jax.numpy as jnp
from jax.experimental import pallas as pl

_B, _S = 8, 4
_T = _B * _S
_HIDDEN = 512
_INTER = 1024
_E = 8
_K = 2


def _moe_kernel(x_ref, rw_ref, w13_ref, w2_ref, out_ref):
    e = pl.program_id(0)
    x = x_ref[...]

    logits = jnp.dot(x, rw_ref[...].T,
                     preferred_element_type=jnp.float32)
    m = jnp.max(logits, axis=-1, keepdims=True)
    ex = jnp.exp(logits - m)
    probs = ex / jnp.sum(ex, axis=-1, keepdims=True)

    cols = jax.lax.broadcasted_iota(jnp.int32, probs.shape, 1)
    i1 = jnp.argmax(probs, axis=-1, keepdims=True)
    v1 = jnp.max(probs, axis=-1)
    masked = jnp.where(cols == i1, -1.0, probs)
    i2 = jnp.argmax(masked, axis=-1, keepdims=True)
    v2 = jnp.max(masked, axis=-1)
    denom = v1 + v2 + 1e-6
    sel = (cols == i1) | (cols == i2)
    dense_w = jnp.where(sel, probs, 0.0) / denom[:, None]
    w_e = jnp.sum(jnp.where(cols == e, dense_w, 0.0), axis=-1)

    h13 = jnp.dot(x, w13_ref[0], preferred_element_type=jnp.float32)
    gate = h13[:, :_INTER]
    up = h13[:, _INTER:]
    h = (gate * jax.nn.sigmoid(gate)) * up
    out_e = jnp.dot(h, w2_ref[0], preferred_element_type=jnp.float32)

    contrib = out_e * w_e[:, None]

    @pl.when(e == 0)
    def _():
        out_ref[...] = contrib

    @pl.when(e != 0)
    def _():
        out_ref[...] = out_ref[...] + contrib


@jax.jit
def kernel(x, router_w, w13, w2):
    xt = x.reshape(_T, _HIDDEN)
    out = pl.pallas_call(
        _moe_kernel,
        grid=(_E,),
        in_specs=[
            pl.BlockSpec((_T, _HIDDEN), lambda e: (0, 0)),
            pl.BlockSpec((_E, _HIDDEN), lambda e: (0, 0)),
            pl.BlockSpec((1, _HIDDEN, 2 * _INTER), lambda e: (e, 0, 0)),
            pl.BlockSpec((1, _INTER, _HIDDEN), lambda e: (e, 0, 0)),
        ],
        out_specs=pl.BlockSpec((_T, _HIDDEN), lambda e: (0, 0)),
        out_shape=jax.ShapeDtypeStruct((_T, _HIDDEN), jnp.float32),
    )(xt, router_w, w13, w2)
    return out.reshape(_B, _S, _HIDDEN)
